# baseline (speedup 1.0000x reference)
import jax
import jax.numpy as jnp
from jax import lax
from jax.experimental import pallas as pl
from jax.experimental.pallas import tpu as pltpu
from jax.experimental.pallas import tpu_sc as plsc

_G0 = 128
_G1 = 128
_E = _G0 + _G1
_K = 4
_NEG = float("-inf")
_NC = 2
_NS = 16
_NW = _NC * _NS
_L = 16


def _matmul_kernel(x_ref, w_ref, b_ref, st_ref):
    st_ref[...] = jax.lax.dot_general(
        w_ref[...], x_ref[...], (((1,), (1,)), ((), ())),
        preferred_element_type=jnp.float32,
    ) + b_ref[...]


def _insert4(vs, ps, c, pc):
    for k in range(_K):
        swap = c > vs[k]
        vs[k], c = jnp.where(swap, c, vs[k]), jnp.where(swap, vs[k], c)
        ps[k], pc = jnp.where(swap, pc, ps[k]), jnp.where(swap, ps[k], pc)


def _sc_route(st_hbm, ids_hbm, lg_hbm, wt_hbm, sc_v, ids_v, lg_v, wt_v):
    tpw = st_hbm.shape[1] // _NW
    wid = lax.axis_index("s") * _NC + lax.axis_index("c")
    base = wid * tpw
    pltpu.sync_copy(st_hbm.at[:, pl.ds(base, tpw)], sc_v)

    neg = jnp.full((_L,), _NEG, jnp.float32)
    zero = jnp.zeros((_L,), jnp.int32)

    def group(g, carry):
        col = pl.ds(g * _L, _L)

        def stage(e0, e1):
            def estep(e, st):
                vs, ps = list(st[:_K]), list(st[_K:])
                _insert4(vs, ps, sc_v[e, col],
                         jnp.broadcast_to(e - e0, (_L,)))
                return (*vs, *ps)
            return lax.fori_loop(
                e0, e1, estep, (neg, neg, neg, neg, zero, zero, zero, zero))

        s0 = stage(0, _G0)
        s1 = stage(_G0, _E)
        v0, i0 = s0[:_K], s0[_K:]
        v1, i1 = s1[:_K], s1[_K:]

        vs, ps = [neg] * _K, [zero] * _K
        for b in range(_K):
            for j in range(_K):
                _insert4(vs, ps, v0[b] + v1[j], i0[b] * _G1 + i1[j])

        es = [jnp.exp(v - vs[0]) for v in vs]
        den = es[0] + es[1] + es[2] + es[3]
        for t in range(_K):
            ids_v[t, col] = ps[t]
            lg_v[t, col] = vs[t]
            wt_v[t, col] = es[t] / den
        return carry

    lax.fori_loop(0, tpw // _L, group, 0)
    pltpu.sync_copy(ids_v, ids_hbm.at[:, pl.ds(base, tpw)])
    pltpu.sync_copy(lg_v, lg_hbm.at[:, pl.ds(base, tpw)])
    pltpu.sync_copy(wt_v, wt_hbm.at[:, pl.ds(base, tpw)])


def kernel(input, W, b):
    n, d = input.shape
    bt = 1024
    chunks = 2
    nc = n // chunks
    b2 = b.reshape(_E, 1)

    matmul = pl.pallas_call(
        _matmul_kernel,
        grid=(nc // bt,),
        in_specs=[
            pl.BlockSpec((bt, d), lambda i: (i, 0)),
            pl.BlockSpec((_E, d), lambda i: (0, 0)),
            pl.BlockSpec((_E, 1), lambda i: (0, 0)),
        ],
        out_specs=pl.BlockSpec((_E, bt), lambda i: (0, i)),
        out_shape=jax.ShapeDtypeStruct((_E, nc), jnp.float32),
    )

    tpw = nc // _NW
    route = pl.kernel(
        _sc_route,
        mesh=plsc.VectorSubcoreMesh(core_axis_name="c", subcore_axis_name="s"),
        out_type=[
            jax.ShapeDtypeStruct((_K, nc), jnp.int32),
            jax.ShapeDtypeStruct((_K, nc), jnp.float32),
            jax.ShapeDtypeStruct((_K, nc), jnp.float32),
        ],
        scratch_types=[
            pltpu.VMEM((_E, tpw), jnp.float32),
            pltpu.VMEM((_K, tpw), jnp.int32),
            pltpu.VMEM((_K, tpw), jnp.float32),
            pltpu.VMEM((_K, tpw), jnp.float32),
        ],
    )

    outs = [route(matmul(input[c * nc:(c + 1) * nc], W, b2))
            for c in range(chunks)]
    ids_t, lg_t, wt_t = (jnp.concatenate(parts, axis=1)
                         for parts in zip(*outs))
    return ids_t.T, lg_t.T, wt_t.T


kernel = jax.jit(kernel)


# speedup vs baseline: 3.4802x; 3.4802x over previous
import jax
import jax.numpy as jnp
from jax.experimental import pallas as pl

_G0 = 128
_G1 = 128
_E = _G0 + _G1
_K = 4
_NEG = float("-inf")


def _top4_rows(x):
    g = x.shape[0]
    iota = jax.lax.broadcasted_iota(jnp.int32, x.shape, 0)
    vals, idxs = [], []
    for _ in range(_K):
        m = jnp.max(x, axis=0, keepdims=True)
        is_max = x == m
        idx = jnp.min(jnp.where(is_max, iota, g), axis=0, keepdims=True)
        vals.append(m)
        idxs.append(idx)
        x = jnp.where(iota == idx, _NEG, x)
    return vals, idxs


def _gating_kernel(x0_ref, x1_ref, w0_ref, w1_ref, b_ref,
                   ids_ref, logits_ref, wts_ref):
    dn = (((1,), (1,)), ((), ()))
    scores = (
        jax.lax.dot_general(w0_ref[...], x0_ref[...], dn,
                            preferred_element_type=jnp.float32)
        + jax.lax.dot_general(w1_ref[...], x1_ref[...], dn,
                              preferred_element_type=jnp.float32)
        + b_ref[...])
    v0, i0 = _top4_rows(scores[:_G0, :])
    v1, i1 = _top4_rows(scores[_G0:, :])

    cand = jnp.concatenate(
        [v0[b] + v1[j] for b in range(_K) for j in range(_K)], axis=0)
    lin = jnp.concatenate(
        [b * _G1 + i1[j] for b in range(_K) for j in range(_K)], axis=0)
    eid = jnp.concatenate(
        [i0[b] * _G1 + i1[j] for b in range(_K) for j in range(_K)], axis=0)

    big = _K * _G1
    ids_rows, logit_rows, exp_rows = [], [], []
    for t in range(_K):
        m = jnp.max(cand, axis=0, keepdims=True)
        l = jnp.min(jnp.where(cand == m, lin, big), axis=0, keepdims=True)
        hit = lin == l
        ids_rows.append(jnp.sum(jnp.where(hit, eid, 0), axis=0, keepdims=True))
        logit_rows.append(m)
        exp_rows.append(jnp.exp(m - logit_rows[0]))
        cand = jnp.where(hit, _NEG, cand)

    denom = exp_rows[0] + exp_rows[1] + exp_rows[2] + exp_rows[3]
    ids_ref[...] = jnp.concatenate(ids_rows, axis=0)
    logits_ref[...] = jnp.concatenate(logit_rows, axis=0)
    wts_ref[...] = jnp.concatenate([e / denom for e in exp_rows], axis=0)


def kernel(input, W, b):
    n, d = input.shape
    bt = 1024
    grid = (n // bt,)
    ids_t, logits_t, wts_t = pl.pallas_call(
        _gating_kernel,
        grid=grid,
        in_specs=[
            pl.BlockSpec((bt, d // 2), lambda i: (i, 0)),
            pl.BlockSpec((bt, d // 2), lambda i: (i, 1)),
            pl.BlockSpec((_E, d // 2), lambda i: (0, 0)),
            pl.BlockSpec((_E, d // 2), lambda i: (0, 1)),
            pl.BlockSpec((_E, 1), lambda i: (0, 0)),
        ],
        out_specs=[
            pl.BlockSpec((_K, bt), lambda i: (0, i)),
            pl.BlockSpec((_K, bt), lambda i: (0, i)),
            pl.BlockSpec((_K, bt), lambda i: (0, i)),
        ],
        out_shape=[
            jax.ShapeDtypeStruct((_K, n), jnp.int32),
            jax.ShapeDtypeStruct((_K, n), jnp.float32),
            jax.ShapeDtypeStruct((_K, n), jnp.float32),
        ],
    )(input, input, W, W, b.reshape(_E, 1))
    return ids_t.T, logits_t.T, wts_t.T


kernel = jax.jit(kernel)
